# Initial kernel scaffold; baseline (speedup 1.0000x reference)
import functools

import jax
import jax.numpy as jnp
from jax import lax
from jax.experimental import pallas as pl
from jax.experimental.pallas import tpu as pltpu
from jax.experimental.pallas import tpu_sc as plsc

NG = 50000
NPNL = 10000
NE = 500000
NPAIR = 100000

NC = 2
NS = 16
NW = NC * NS

NBE = 16
NEP = NW * NBE * 1024

ACC_P = 10240
ACC_G = 50176

NPB = (NPAIR + 511) // 512
NPP = NPB * 512

_mesh = plsc.VectorSubcoreMesh(core_axis_name="c", subcore_axis_name="s")
_f32 = jnp.float32


def _worker_id():
    return lax.axis_index("c") * NS + lax.axis_index("s")


def _zero_fill(zbuf, width):
    @pl.loop(0, 32)
    def _(i):
        for c in range(width // 16):
            zbuf[i, pl.ds(c * 16, 16)] = jnp.zeros((16,), _f32)


def _counts_body(dp_hbm, dg_hbm, cntp_hbm, cntg_hbm,
                 idxp, idxg, ones, zbuf, accp, accg, sem):
    del sem
    sid = lax.axis_index("s")
    cid = lax.axis_index("c")
    w = _worker_id()

    @pl.loop(0, 128)
    def _(i):
        ones[i, :] = jnp.ones((16,), _f32)
    _zero_fill(zbuf, 16)

    @pl.loop(0, ACC_P // (16 * 32))
    def _(i):
        pltpu.sync_copy(zbuf, accp.at[pl.ds(sid * (ACC_P // 16) + i * 32, 32)])

    @pl.loop(0, ACC_G // (16 * 32))
    def _(i):
        pltpu.sync_copy(zbuf, accg.at[pl.ds(sid * (ACC_G // 16) + i * 32, 32)])

    plsc.subcore_barrier()

    rowbase = w * (NBE * 8)

    @pl.loop(0, NBE)
    def _(b):
        r0 = rowbase + b * 8
        pltpu.sync_copy(dp_hbm.at[pl.ds(r0, 8)], idxp)
        pltpu.sync_copy(dg_hbm.at[pl.ds(r0, 8)], idxg)
        for j in range(8):
            pltpu.sync_copy(ones, accp.at[idxp.at[j]], add=True)
            pltpu.sync_copy(ones, accg.at[idxg.at[j]], add=True)

    plsc.subcore_barrier()
    pltpu.sync_copy(accp.at[pl.ds(sid * (NPNL // 16), NPNL // 16)],
                    cntp_hbm.at[cid, pl.ds(sid * (NPNL // 16), NPNL // 16)])
    pltpu.sync_copy(accg.at[pl.ds(sid * (NG // 16), NG // 16)],
                    cntg_hbm.at[cid, pl.ds(sid * (NG // 16), NG // 16)])


_counts_call = pl.kernel(
    _counts_body,
    out_type=[jax.ShapeDtypeStruct((NC, NPNL, 16), _f32),
              jax.ShapeDtypeStruct((NC, NG, 16), _f32)],
    mesh=_mesh,
    scratch_types=[
        pltpu.VMEM((8, 128), jnp.int32),
        pltpu.VMEM((8, 128), jnp.int32),
        pltpu.VMEM((128, 16), _f32),
        pltpu.VMEM((32, 16), _f32),
        pltpu.VMEM_SHARED((ACC_P, 16), _f32),
        pltpu.VMEM_SHARED((ACC_G, 16), _f32),
        pltpu.SemaphoreType.DMA,
    ],
)


def _aggp_body(tbl_hbm, src_hbm, dst_hbm, out_hbm,
               idxs, idxd, rows, zbuf, acc, sem):
    sid = lax.axis_index("s")
    cid = lax.axis_index("c")
    w = _worker_id()

    _zero_fill(zbuf, 128)

    @pl.loop(0, ACC_P // (16 * 32))
    def _(i):
        pltpu.sync_copy(zbuf, acc.at[pl.ds(sid * (ACC_P // 16) + i * 32, 32)])

    plsc.subcore_barrier()
    rowbase = w * (NBE * 8)

    @pl.loop(0, NBE)
    def _(b):
        r0 = rowbase + b * 8
        pltpu.sync_copy(src_hbm.at[pl.ds(r0, 8)], idxs)
        pltpu.sync_copy(dst_hbm.at[pl.ds(r0, 8)], idxd)
        for j in range(8):
            pltpu.async_copy(tbl_hbm.at[idxs.at[j]], rows, sem).wait()
            pltpu.sync_copy(rows, acc.at[idxd.at[j]], add=True)

    plsc.subcore_barrier()
    pltpu.sync_copy(acc.at[pl.ds(sid * (NPNL // 16), NPNL // 16)],
                    out_hbm.at[cid, pl.ds(sid * (NPNL // 16), NPNL // 16)])


_aggp_call = pl.kernel(
    _aggp_body,
    out_type=jax.ShapeDtypeStruct((NC, NPNL, 128), _f32),
    mesh=_mesh,
    scratch_types=[
        pltpu.VMEM((8, 128), jnp.int32),
        pltpu.VMEM((8, 128), jnp.int32),
        pltpu.VMEM((128, 128), _f32),
        pltpu.VMEM((32, 128), _f32),
        pltpu.VMEM_SHARED((ACC_P, 128), _f32),
        pltpu.SemaphoreType.DMA,
    ],
)


def _aggg_body(t0, t1, t2, t3, src_hbm, dst_hbm, out_hbm,
               idxs, idxd, rows, zbuf, acc, sem):
    sid = lax.axis_index("s")
    cid = lax.axis_index("c")
    w = _worker_id()
    rowbase = w * (NBE * 8)

    _zero_fill(zbuf, 32)

    for q, tbl in enumerate((t0, t1, t2, t3)):
        @pl.loop(0, ACC_G // (16 * 32))
        def _(i):
            pltpu.sync_copy(zbuf,
                            acc.at[pl.ds(sid * (ACC_G // 16) + i * 32, 32)])

        plsc.subcore_barrier()

        @pl.loop(0, NBE)
        def _(b):
            r0 = rowbase + b * 8
            pltpu.sync_copy(src_hbm.at[pl.ds(r0, 8)], idxs)
            pltpu.sync_copy(dst_hbm.at[pl.ds(r0, 8)], idxd)
            for j in range(8):
                pltpu.async_copy(tbl.at[idxs.at[j]], rows, sem).wait()
                pltpu.sync_copy(rows, acc.at[idxd.at[j]], add=True)

        plsc.subcore_barrier()
        pltpu.sync_copy(acc.at[pl.ds(sid * (NG // 16), NG // 16)],
                        out_hbm.at[cid, q, pl.ds(sid * (NG // 16), NG // 16)])
        plsc.subcore_barrier()


_aggg_call = pl.kernel(
    _aggg_body,
    out_type=jax.ShapeDtypeStruct((NC, 4, NG, 32), _f32),
    mesh=_mesh,
    scratch_types=[
        pltpu.VMEM((8, 128), jnp.int32),
        pltpu.VMEM((8, 128), jnp.int32),
        pltpu.VMEM((128, 32), _f32),
        pltpu.VMEM((32, 32), _f32),
        pltpu.VMEM_SHARED((ACC_G, 32), _f32),
        pltpu.SemaphoreType.DMA,
    ],
)


def _pairs_body(g_hbm, p_hbm, pg_hbm, pp_hbm, go_hbm, po_hbm,
                idx, rows, sem):
    w = _worker_id()
    nb = jnp.where(w < NPB - (NPB // NW) * NW, NPB // NW + 1, NPB // NW)

    @pl.loop(0, nb)
    def _(k):
        b = w + k * NW
        pltpu.sync_copy(pg_hbm.at[pl.ds(b * 4, 4)], idx)
        for j in range(4):
            pltpu.async_copy(g_hbm.at[idx.at[j]],
                             rows.at[pl.ds(j * 128, 128)], sem).wait()
        pltpu.sync_copy(rows, go_hbm.at[pl.ds(b * 512, 512)])
        pltpu.sync_copy(pp_hbm.at[pl.ds(b * 4, 4)], idx)
        for j in range(4):
            pltpu.async_copy(p_hbm.at[idx.at[j]],
                             rows.at[pl.ds(j * 128, 128)], sem).wait()
        pltpu.sync_copy(rows, po_hbm.at[pl.ds(b * 512, 512)])


_pairs_call = pl.kernel(
    _pairs_body,
    out_type=[jax.ShapeDtypeStruct((NPP, 128), _f32),
              jax.ShapeDtypeStruct((NPP, 128), _f32)],
    mesh=_mesh,
    scratch_types=[
        pltpu.VMEM((4, 128), jnp.int32),
        pltpu.VMEM((512, 128), _f32),
        pltpu.SemaphoreType.DMA,
    ],
)


def _dot_t(a, w):
    return lax.dot_general(a, w, dimension_numbers=(((1,), (1,)), ((), ())),
                           preferred_element_type=_f32)


def _sage_p_body(relu, parts, cnt, x, wl, wr, b, o):
    c = cnt[0, :, 0] + cnt[1, :, 0]
    agg = (parts[0] + parts[1]) / jnp.maximum(c, 1.0)[:, None]
    r = _dot_t(agg, wl[...]) + b[...] + _dot_t(x[...], wr[...])
    o[...] = jnp.maximum(r, 0.0) if relu else r


def _sage_g_body(relu, parts, cnt, x, wl, wr, b, o):
    c = cnt[0, :, 0] + cnt[1, :, 0]
    pq = parts[0] + parts[1]
    agg = jnp.concatenate([pq[0], pq[1], pq[2], pq[3]], axis=1)
    agg = agg / jnp.maximum(c, 1.0)[:, None]
    r = _dot_t(agg, wl[...]) + b[...] + _dot_t(x[...], wr[...])
    o[...] = jnp.maximum(r, 0.0) if relu else r


def _sage_p(parts, cnt, x, wl, wr, b, relu):
    bn = 1000
    return pl.pallas_call(
        functools.partial(_sage_p_body, relu),
        grid=(NPNL // bn,),
        in_specs=[
            pl.BlockSpec((NC, bn, 128), lambda i: (0, i, 0)),
            pl.BlockSpec((NC, bn, 16), lambda i: (0, i, 0)),
            pl.BlockSpec((bn, 128), lambda i: (i, 0)),
            pl.BlockSpec((128, 128), lambda i: (0, 0)),
            pl.BlockSpec((128, 128), lambda i: (0, 0)),
            pl.BlockSpec((1, 128), lambda i: (0, 0)),
        ],
        out_specs=pl.BlockSpec((bn, 128), lambda i: (i, 0)),
        out_shape=jax.ShapeDtypeStruct((NPNL, 128), _f32),
    )(parts, cnt, x, wl, wr, b)


def _sage_g(parts, cnt, x, wl, wr, b, relu):
    bn = 1000
    return pl.pallas_call(
        functools.partial(_sage_g_body, relu),
        grid=(NG // bn,),
        in_specs=[
            pl.BlockSpec((NC, 4, bn, 32), lambda i: (0, 0, i, 0)),
            pl.BlockSpec((NC, bn, 16), lambda i: (0, i, 0)),
            pl.BlockSpec((bn, 128), lambda i: (i, 0)),
            pl.BlockSpec((128, 128), lambda i: (0, 0)),
            pl.BlockSpec((128, 128), lambda i: (0, 0)),
            pl.BlockSpec((1, 128), lambda i: (0, 0)),
        ],
        out_specs=pl.BlockSpec((bn, 128), lambda i: (i, 0)),
        out_shape=jax.ShapeDtypeStruct((NG, 128), _f32),
    )(parts, cnt, x, wl, wr, b)


def _proj_body(x, w, o):
    o[...] = _dot_t(x[...], w[...])


def _proj(x, w):
    bn = 1000
    return pl.pallas_call(
        _proj_body,
        grid=(NPNL // bn,),
        in_specs=[pl.BlockSpec((bn, 128), lambda i: (i, 0)),
                  pl.BlockSpec((128, 128), lambda i: (0, 0))],
        out_specs=pl.BlockSpec((bn, 128), lambda i: (i, 0)),
        out_shape=jax.ShapeDtypeStruct((NPNL, 128), _f32),
    )(x, w)


def _score_body(g, p, bb, o):
    o[...] = jnp.sum(g[...] * p[...], axis=1) + bb[0, 0]


def _score(gr, pr, bb):
    bn = 2048
    return pl.pallas_call(
        _score_body,
        grid=(NPP // bn,),
        in_specs=[pl.BlockSpec((bn, 128), lambda i: (i, 0)),
                  pl.BlockSpec((bn, 128), lambda i: (i, 0)),
                  pl.BlockSpec((1, 1), lambda i: (0, 0))],
        out_specs=pl.BlockSpec((bn,), lambda i: (i,)),
        out_shape=jax.ShapeDtypeStruct((NPP,), _f32),
    )(gr, pr, bb)


def _pad_idx(a, n, fill):
    a = a.astype(jnp.int32)
    return jnp.concatenate(
        [a, jnp.full((n - a.shape[0],), fill, jnp.int32)]).reshape(-1, 128)


def kernel(x_gene, x_panel, edge_gene, edge_panel, pairs_gene, pairs_panel,
           W_l1_p, b_l1_p, W_r1_p, W_l1_g, b_l1_g, W_r1_g,
           W_l2_p, b_l2_p, W_r2_p, W_l2_g, b_l2_g, W_r2_g, W_bil, b_bil):
    src_b = _pad_idx(edge_gene, NEP, 0)
    dst_b = _pad_idx(edge_panel, NEP, NPNL)
    src_c = _pad_idx(edge_panel, NEP, 0)
    dst_c = _pad_idx(edge_gene, NEP, NG)
    pg = _pad_idx(pairs_gene, NPP, 0)
    pp = _pad_idx(pairs_panel, NPP, 0)

    cntp, cntg = _counts_call(dst_b, dst_c)

    def quarters(x):
        return tuple(x[:, q * 32:(q + 1) * 32] for q in range(4))

    b1p = b_l1_p.reshape(1, 128)
    b1g = b_l1_g.reshape(1, 128)
    b2p = b_l2_p.reshape(1, 128)
    b2g = b_l2_g.reshape(1, 128)

    aggp1 = _aggp_call(x_gene, src_b, dst_b)
    aggg1 = _aggg_call(*quarters(x_panel), src_c, dst_c)
    p1 = _sage_p(aggp1, cntp, x_panel, W_l1_p, W_r1_p, b1p, relu=True)
    g1 = _sage_g(aggg1, cntg, x_gene, W_l1_g, W_r1_g, b1g, relu=True)

    aggp2 = _aggp_call(g1, src_b, dst_b)
    aggg2 = _aggg_call(*quarters(p1), src_c, dst_c)
    p2 = _sage_p(aggp2, cntp, p1, W_l2_p, W_r2_p, b2p, relu=False)
    g2 = _sage_g(aggg2, cntg, g1, W_l2_g, W_r2_g, b2g, relu=False)

    p2t = _proj(p2, W_bil[0])
    gr, pr = _pairs_call(g2, p2t, pg, pp)
    s = _score(gr, pr, b_bil.reshape(1, 1))
    return s[:NPAIR]


# speedup vs baseline: 1.2298x; 1.2298x over previous
import functools

import jax
import jax.numpy as jnp
from jax import lax
from jax.experimental import pallas as pl
from jax.experimental.pallas import tpu as pltpu
from jax.experimental.pallas import tpu_sc as plsc

NG = 50000
NPNL = 10000
NE = 500000
NPAIR = 100000

NC = 2
NS = 16
NW = NC * NS

NBE = 16
NEP = NW * NBE * 1024

ACC_P = 10240
ACC_G = 50176

NPB = (NPAIR + 1023) // 1024
NPP = NPB * 1024

_mesh = plsc.VectorSubcoreMesh(core_axis_name="c", subcore_axis_name="s")
_f32 = jnp.float32
_sc_params = pltpu.CompilerParams(needs_layout_passes=False)


def _worker_id():
    return lax.axis_index("c") * NS + lax.axis_index("s")


HP_N = 10240
HG_N = 50176

def _counts_body(dpf_hbm, dgf_hbm, cntp_hbm, cntg_hbm,
                 idxp, idxg, hp, hg, sem):
    del sem
    w = _worker_id()
    z = jnp.zeros((16,), _f32)
    one = jnp.ones((16,), _f32)

    @pl.loop(0, HP_N // 16)
    def _(i):
        hp[pl.ds(i * 16, 16)] = z

    @pl.loop(0, HG_N // 16)
    def _(i):
        hg[pl.ds(i * 16, 16)] = z

    ebase = w * (NBE * 1024)

    @pl.loop(0, NBE)
    def _(b):
        eo = ebase + b * 1024
        pltpu.sync_copy(dpf_hbm.at[pl.ds(eo, 1024)], idxp)
        pltpu.sync_copy(dgf_hbm.at[pl.ds(eo, 1024)], idxg)

        @pl.loop(0, 64)
        def _(g):
            plsc.addupdate_scatter(hp, [idxp[pl.ds(g * 16, 16)]], one)
            plsc.addupdate_scatter(hg, [idxg[pl.ds(g * 16, 16)]], one)

    pltpu.sync_copy(hp, cntp_hbm.at[pl.ds(w * HP_N, HP_N)])
    pltpu.sync_copy(hg, cntg_hbm.at[pl.ds(w * HG_N, HG_N)])


_counts_call = pl.kernel(
    _counts_body,
    out_type=[jax.ShapeDtypeStruct((NW * HP_N,), _f32),
              jax.ShapeDtypeStruct((NW * HG_N,), _f32)],
    mesh=_mesh,
    scratch_types=[
        pltpu.VMEM((1024,), jnp.int32),
        pltpu.VMEM((1024,), jnp.int32),
        pltpu.VMEM((HP_N,), _f32),
        pltpu.VMEM((HG_N,), _f32),
        pltpu.SemaphoreType.DMA,
    ],
    compiler_params=_sc_params,
)


def _aggp_body(tbl_hbm, src_hbm, dst_hbm, zro_hbm, out_hbm,
               idxs, idxd, rows, acc, sem):
    sid = lax.axis_index("s")
    cid = lax.axis_index("c")
    w = _worker_id()

    pltpu.sync_copy(zro_hbm.at[pl.ds(0, ACC_P // 16)],
                    acc.at[pl.ds(sid * (ACC_P // 16), ACC_P // 16)])
    plsc.subcore_barrier()
    rowbase = w * (NBE * 8)

    @pl.loop(0, NBE)
    def _(b):
        r0 = rowbase + b * 8
        pltpu.sync_copy(src_hbm.at[pl.ds(r0, 8)], idxs)
        pltpu.sync_copy(dst_hbm.at[pl.ds(r0, 8)], idxd)
        for j in range(8):
            pltpu.async_copy(tbl_hbm.at[idxs.at[j]], rows, sem).wait()
            pltpu.sync_copy(rows, acc.at[idxd.at[j]], add=True)

    plsc.subcore_barrier()

    @pl.when(sid < 10)
    def _():
        o = pl.multiple_of(sid * 1000, 8)
        pltpu.sync_copy(acc.at[pl.ds(o, 1000)],
                        out_hbm.at[cid, pl.ds(o, 1000)])


_aggp_call = pl.kernel(
    _aggp_body,
    out_type=jax.ShapeDtypeStruct((NC, NPNL, 128), _f32),
    mesh=_mesh,
    scratch_types=[
        pltpu.VMEM((8, 128), jnp.int32),
        pltpu.VMEM((8, 128), jnp.int32),
        pltpu.VMEM((128, 128), _f32),
        pltpu.VMEM_SHARED((ACC_P, 128), _f32),
        pltpu.SemaphoreType.DMA,
    ],
)


RNG_G = 12544
ACC_GR = 12800
NSUB = 4
SBLK = NBE // NSUB
NCMP = SBLK * 1024 + 128

def _aggg_body(tbl_hbm, srcf_hbm, dstf_hbm, zro_hbm, out_hbm,
               idxs, idxd, csrc, cdst, dtmp, rows, acc, sem):
    sid = lax.axis_index("s")
    cid = lax.axis_index("c")
    w = _worker_id()
    ebase = w * (NBE * 1024)

    zi = jnp.zeros((16,), jnp.int32)
    di = jnp.full((16,), RNG_G, jnp.int32)

    for p in range(4):
        r0 = p * RNG_G

        pltpu.sync_copy(zro_hbm, acc.at[pl.ds(sid * (ACC_GR // 16),
                                              ACC_GR // 16)])
        plsc.subcore_barrier()

        @pl.loop(0, NSUB)
        def _(sub):
            def blk(b, n):
                eo = ebase + (sub * SBLK + b) * 1024
                pltpu.sync_copy(srcf_hbm.at[pl.ds(eo, 1024)], idxs)
                pltpu.sync_copy(dstf_hbm.at[pl.ds(eo, 1024)], idxd)

                def grp(g, n):
                    dv = idxd[pl.ds(g * 16, 16)] - r0
                    sv = idxs[pl.ds(g * 16, 16)]
                    m = (dv >= 0) & (dv < RNG_G)
                    cum = plsc.cumsum(jnp.where(m, 1, 0))
                    pos = n + cum - 1
                    plsc.store_scatter(csrc, [pos], sv, mask=m)
                    plsc.store_scatter(cdst, [pos], dv, mask=m)
                    return n + jnp.max(cum)

                return pl.loop(0, 64, init_carry=n)(grp)

            n = pl.loop(0, SBLK, init_carry=jnp.int32(0))(blk)

            for t in range(8):
                csrc[pl.ds(n + t * 16, 16)] = zi
                cdst[pl.ds(n + t * 16, 16)] = di

            @pl.loop(0, (n + 127) // 128)
            def _(k):
                pltpu.async_copy(tbl_hbm.at[csrc.at[pl.ds(k * 128, 128)]],
                                 rows, sem).wait()
                for c in range(8):
                    dtmp[pl.ds(c * 16, 16)] = cdst[pl.ds(k * 128 + c * 16,
                                                         16)]
                pltpu.sync_copy(rows, acc.at[dtmp], add=True)

        plsc.subcore_barrier()
        o = pl.multiple_of(sid * (RNG_G // 16), 8)
        pltpu.sync_copy(acc.at[pl.ds(o, RNG_G // 16)],
                        out_hbm.at[cid, pl.ds(r0 + o, RNG_G // 16)])
        plsc.subcore_barrier()


_aggg_call = pl.kernel(
    _aggg_body,
    out_type=jax.ShapeDtypeStruct((NC, 4 * RNG_G, 128), _f32),
    mesh=_mesh,
    scratch_types=[
        pltpu.VMEM((1024,), jnp.int32),
        pltpu.VMEM((1024,), jnp.int32),
        pltpu.VMEM((NCMP,), jnp.int32),
        pltpu.VMEM((NCMP,), jnp.int32),
        pltpu.VMEM((128,), jnp.int32),
        pltpu.VMEM((128, 128), _f32),
        pltpu.VMEM_SHARED((ACC_GR, 128), _f32),
        pltpu.SemaphoreType.DMA,
    ],
    compiler_params=_sc_params,
)


def _pairs_body(g_hbm, p_hbm, pg_hbm, pp_hbm, go_hbm, po_hbm,
                idx, rows, sem):
    w = _worker_id()
    nb = jnp.where(w < NPB - (NPB // NW) * NW, NPB // NW + 1, NPB // NW)

    @pl.loop(0, nb)
    def _(k):
        b = w + k * NW
        for tbl, ihbm, ohbm in ((g_hbm, pg_hbm, go_hbm),
                                (p_hbm, pp_hbm, po_hbm)):
            pltpu.sync_copy(ihbm.at[pl.ds(b * 8, 8)], idx)
            for h in range(2):
                for j in range(4):
                    pltpu.async_copy(tbl.at[idx.at[h * 4 + j]],
                                     rows.at[pl.ds(j * 128, 128)], sem).wait()
                o = pl.multiple_of(b * 1024 + h * 512, 8)
                pltpu.sync_copy(rows, ohbm.at[pl.ds(o, 512)])


_pairs_call = pl.kernel(
    _pairs_body,
    out_type=[jax.ShapeDtypeStruct((NPP, 128), _f32),
              jax.ShapeDtypeStruct((NPP, 128), _f32)],
    mesh=_mesh,
    scratch_types=[
        pltpu.VMEM((8, 128), jnp.int32),
        pltpu.VMEM((512, 128), _f32),
        pltpu.SemaphoreType.DMA,
    ],
)


def _dot_t(a, w):
    return lax.dot_general(a, w, dimension_numbers=(((1,), (1,)), ((), ())),
                           preferred_element_type=_f32)


def _cntsum_body(cin, o):
    c = jnp.sum(cin[...], axis=0)
    o[...] = (1.0 / jnp.maximum(c, 1.0))[:, None] * jnp.ones((1, 128), _f32)


def _cntsum(cflat, hn):
    bn = 1024
    return pl.pallas_call(
        _cntsum_body,
        grid=(hn // bn,),
        in_specs=[pl.BlockSpec((NW, bn), lambda i: (0, i))],
        out_specs=pl.BlockSpec((bn, 128), lambda i: (i, 0)),
        out_shape=jax.ShapeDtypeStruct((hn, 128), _f32),
    )(cflat.reshape(NW, hn))


def _sage_body(relu, parts, rcp, x, wl, wr, b, o):
    agg = (parts[0] + parts[1]) * rcp[...]
    r = _dot_t(agg, wl[...]) + b[...] + _dot_t(x[...], wr[...])
    o[...] = jnp.maximum(r, 0.0) if relu else r


def _sage(parts, rcp, x, wl, wr, b, relu, n):
    bn = 1000
    return pl.pallas_call(
        functools.partial(_sage_body, relu),
        grid=(n // bn,),
        in_specs=[
            pl.BlockSpec((NC, bn, 128), lambda i: (0, i, 0)),
            pl.BlockSpec((bn, 128), lambda i: (i, 0)),
            pl.BlockSpec((bn, 128), lambda i: (i, 0)),
            pl.BlockSpec((128, 128), lambda i: (0, 0)),
            pl.BlockSpec((128, 128), lambda i: (0, 0)),
            pl.BlockSpec((1, 128), lambda i: (0, 0)),
        ],
        out_specs=pl.BlockSpec((bn, 128), lambda i: (i, 0)),
        out_shape=jax.ShapeDtypeStruct((n, 128), _f32),
    )(parts, rcp, x, wl, wr, b)


def _proj_body(x, w, o):
    o[...] = _dot_t(x[...], w[...])


def _proj(x, w):
    bn = 1000
    return pl.pallas_call(
        _proj_body,
        grid=(NPNL // bn,),
        in_specs=[pl.BlockSpec((bn, 128), lambda i: (i, 0)),
                  pl.BlockSpec((128, 128), lambda i: (0, 0))],
        out_specs=pl.BlockSpec((bn, 128), lambda i: (i, 0)),
        out_shape=jax.ShapeDtypeStruct((NPNL, 128), _f32),
    )(x, w)


def _score_body(g, p, bb, o):
    o[...] = jnp.sum(g[...] * p[...], axis=1) + bb[0, 0]


def _score(gr, pr, bb):
    bn = 2048
    return pl.pallas_call(
        _score_body,
        grid=(NPP // bn,),
        in_specs=[pl.BlockSpec((bn, 128), lambda i: (i, 0)),
                  pl.BlockSpec((bn, 128), lambda i: (i, 0)),
                  pl.BlockSpec((1, 1), lambda i: (0, 0))],
        out_specs=pl.BlockSpec((bn,), lambda i: (i,)),
        out_shape=jax.ShapeDtypeStruct((NPP,), _f32),
    )(gr, pr, bb)


def _pad_idx(a, n, fill):
    a = a.astype(jnp.int32)
    return jnp.concatenate([a, jnp.full((n - a.shape[0],), fill, jnp.int32)])


def kernel(x_gene, x_panel, edge_gene, edge_panel, pairs_gene, pairs_panel,
           W_l1_p, b_l1_p, W_r1_p, W_l1_g, b_l1_g, W_r1_g,
           W_l2_p, b_l2_p, W_r2_p, W_l2_g, b_l2_g, W_r2_g, W_bil, b_bil):
    src_b = _pad_idx(edge_gene, NEP, 0).reshape(-1, 128)
    dst_bf = _pad_idx(edge_panel, NEP, NPNL)
    dst_b = dst_bf.reshape(-1, 128)
    src_cf = _pad_idx(edge_panel, NEP, 0)
    dst_cf = _pad_idx(edge_gene, NEP, NG)
    pg = _pad_idx(pairs_gene, NPP, 0).reshape(-1, 128)
    pp = _pad_idx(pairs_panel, NPP, 0).reshape(-1, 128)
    zro = jnp.zeros((ACC_GR // 16, 128), _f32)

    cpf, cgf = _counts_call(dst_bf, dst_cf)
    rcpp = _cntsum(cpf, HP_N)
    rcpg = _cntsum(cgf, HG_N)

    b1p = b_l1_p.reshape(1, 128)
    b1g = b_l1_g.reshape(1, 128)
    b2p = b_l2_p.reshape(1, 128)
    b2g = b_l2_g.reshape(1, 128)

    aggp1 = _aggp_call(x_gene, src_b, dst_b, zro)
    aggg1 = _aggg_call(x_panel, src_cf, dst_cf, zro)
    p1 = _sage(aggp1, rcpp[:NPNL], x_panel, W_l1_p, W_r1_p, b1p, True, NPNL)
    g1 = _sage(aggg1, rcpg[:NG], x_gene, W_l1_g, W_r1_g, b1g, True, NG)

    aggp2 = _aggp_call(g1, src_b, dst_b, zro)
    aggg2 = _aggg_call(p1, src_cf, dst_cf, zro)
    p2 = _sage(aggp2, rcpp[:NPNL], p1, W_l2_p, W_r2_p, b2p, False, NPNL)
    g2 = _sage(aggg2, rcpg[:NG], g1, W_l2_g, W_r2_g, b2g, False, NG)

    p2t = _proj(p2, W_bil[0])
    gr, pr = _pairs_call(g2, p2t, pg, pp)
    s = _score(gr, pr, b_bil.reshape(1, 1))
    return s[:NPAIR]


# speedup vs baseline: 1.4167x; 1.1520x over previous
import functools

import jax
import jax.numpy as jnp
from jax import lax
from jax.experimental import pallas as pl
from jax.experimental.pallas import tpu as pltpu
from jax.experimental.pallas import tpu_sc as plsc

NG = 50000
NPNL = 10000
NE = 500000
NPAIR = 100000

NC = 2
NS = 16
NW = NC * NS

NBE = 16
NEP = NW * NBE * 1024

ACC_P = 10240
ACC_G = 50176

NPB = (NPAIR + 1023) // 1024
NPP = NPB * 1024

_mesh = plsc.VectorSubcoreMesh(core_axis_name="c", subcore_axis_name="s")
_f32 = jnp.float32
_sc_params = pltpu.CompilerParams(needs_layout_passes=False)


def _worker_id():
    return lax.axis_index("c") * NS + lax.axis_index("s")


HP_N = 10240
HG_N = 50176

def _counts_body(dpf_hbm, dgf_hbm, cntp_hbm, cntg_hbm,
                 idxp, idxg, hp, hg, sem):
    del sem
    w = _worker_id()
    z = jnp.zeros((16,), _f32)
    one = jnp.ones((16,), _f32)

    @pl.loop(0, HP_N // 16)
    def _(i):
        hp[pl.ds(i * 16, 16)] = z

    @pl.loop(0, HG_N // 16)
    def _(i):
        hg[pl.ds(i * 16, 16)] = z

    ebase = w * (NBE * 1024)

    @pl.loop(0, NBE)
    def _(b):
        eo = ebase + b * 1024
        pltpu.sync_copy(dpf_hbm.at[pl.ds(eo, 1024)], idxp)
        pltpu.sync_copy(dgf_hbm.at[pl.ds(eo, 1024)], idxg)

        @pl.loop(0, 64)
        def _(g):
            plsc.addupdate_scatter(hp, [idxp[pl.ds(g * 16, 16)]], one)
            plsc.addupdate_scatter(hg, [idxg[pl.ds(g * 16, 16)]], one)

    pltpu.sync_copy(hp, cntp_hbm.at[pl.ds(w * HP_N, HP_N)])
    pltpu.sync_copy(hg, cntg_hbm.at[pl.ds(w * HG_N, HG_N)])


_counts_call = pl.kernel(
    _counts_body,
    out_type=[jax.ShapeDtypeStruct((NW * HP_N,), _f32),
              jax.ShapeDtypeStruct((NW * HG_N,), _f32)],
    mesh=_mesh,
    scratch_types=[
        pltpu.VMEM((1024,), jnp.int32),
        pltpu.VMEM((1024,), jnp.int32),
        pltpu.VMEM((HP_N,), _f32),
        pltpu.VMEM((HG_N,), _f32),
        pltpu.SemaphoreType.DMA,
    ],
    compiler_params=_sc_params,
)


CH = 64
NRB = 4

def _cpy_idx(dst, src, off):
    for c in range(CH // 16):
        dst[pl.ds(c * 16, 16)] = src[pl.ds(off + c * 16, 16)]


def _aggp_body(tbl_hbm, srcf_hbm, dstf_hbm, zro_hbm, out_hbm,
               idxs, idxd, dtmp, r0b, r1b, r2b, r3b,
               s0, s1, s2, s3, acc):
    sid = lax.axis_index("s")
    cid = lax.axis_index("c")
    w = _worker_id()
    rbufs = (r0b, r1b, r2b, r3b)
    sems = (s0, s1, s2, s3)
    nch = 1024 // CH

    pltpu.sync_copy(zro_hbm.at[pl.ds(0, ACC_P // 16)],
                    acc.at[pl.ds(sid * (ACC_P // 16), ACC_P // 16)])
    plsc.subcore_barrier()
    ebase = w * (NBE * 1024)

    def fire(k, t):
        pltpu.async_copy(tbl_hbm.at[idxs.at[pl.ds(k * CH, CH)]],
                         rbufs[t], sems[t])

    def drain(t):
        pltpu.make_async_copy(tbl_hbm.at[pl.ds(0, CH)], rbufs[t],
                              sems[t]).wait()

    @pl.loop(0, NBE)
    def _(b):
        eo = ebase + b * 1024
        pltpu.sync_copy(srcf_hbm.at[pl.ds(eo, 1024)], idxs)
        pltpu.sync_copy(dstf_hbm.at[pl.ds(eo, 1024)], idxd)
        for t in range(NRB - 1):
            fire(t, t)
        for k in range(nch):
            t = k % NRB
            drain(t)
            if k + NRB - 1 < nch:
                fire(k + NRB - 1, (k + NRB - 1) % NRB)
            _cpy_idx(dtmp, idxd, k * CH)
            pltpu.sync_copy(rbufs[t], acc.at[dtmp], add=True)

    plsc.subcore_barrier()

    @pl.when(sid < 10)
    def _():
        o = pl.multiple_of(sid * 1000, 8)
        pltpu.sync_copy(acc.at[pl.ds(o, 1000)],
                        out_hbm.at[cid, pl.ds(o, 1000)])


_aggp_call = pl.kernel(
    _aggp_body,
    out_type=jax.ShapeDtypeStruct((NC, NPNL, 128), _f32),
    mesh=_mesh,
    scratch_types=[
        pltpu.VMEM((1024,), jnp.int32),
        pltpu.VMEM((1024,), jnp.int32),
        pltpu.VMEM((CH,), jnp.int32),
        pltpu.VMEM((CH, 128), _f32),
        pltpu.VMEM((CH, 128), _f32),
        pltpu.VMEM((CH, 128), _f32),
        pltpu.VMEM((CH, 128), _f32),
        pltpu.SemaphoreType.DMA,
        pltpu.SemaphoreType.DMA,
        pltpu.SemaphoreType.DMA,
        pltpu.SemaphoreType.DMA,
        pltpu.VMEM_SHARED((ACC_P, 128), _f32),
    ],
    compiler_params=_sc_params,
)


RNG_G = 10112
NPASS = 5
ACC_GR = 10240
GOUT = NPASS * RNG_G
NSUB = 4
SBLK = NBE // NSUB
NCMP = SBLK * 1024 + 128

def _aggg_body(tbl_hbm, srcf_hbm, dstf_hbm, zro_hbm, out_hbm,
               idxs, idxd, csrc, cdst, dtmp, r0b, r1b, r2b, r3b,
               s0, s1, s2, s3, acc):
    sid = lax.axis_index("s")
    cid = lax.axis_index("c")
    w = _worker_id()
    ebase = w * (NBE * 1024)
    rbufs = (r0b, r1b, r2b, r3b)
    sems = (s0, s1, s2, s3)

    zi = jnp.zeros((16,), jnp.int32)
    di = jnp.full((16,), RNG_G, jnp.int32)

    def fire(k, t):
        pltpu.async_copy(tbl_hbm.at[csrc.at[pl.ds(k * CH, CH)]],
                         rbufs[t], sems[t])

    def drain(t):
        pltpu.make_async_copy(tbl_hbm.at[pl.ds(0, CH)], rbufs[t],
                              sems[t]).wait()

    for p in range(NPASS):
        r0 = p * RNG_G

        pltpu.sync_copy(zro_hbm.at[pl.ds(0, ACC_GR // 16)],
                        acc.at[pl.ds(sid * (ACC_GR // 16), ACC_GR // 16)])
        plsc.subcore_barrier()

        @pl.loop(0, NSUB)
        def _(sub):
            def blk(b, n):
                eo = ebase + (sub * SBLK + b) * 1024
                pltpu.sync_copy(srcf_hbm.at[pl.ds(eo, 1024)], idxs)
                pltpu.sync_copy(dstf_hbm.at[pl.ds(eo, 1024)], idxd)

                def grp(g, n):
                    dv = idxd[pl.ds(g * 16, 16)] - r0
                    sv = idxs[pl.ds(g * 16, 16)]
                    m = (dv >= 0) & (dv < RNG_G)
                    cum = plsc.cumsum(jnp.where(m, 1, 0))
                    pos = n + cum - 1
                    plsc.store_scatter(csrc, [pos], sv, mask=m)
                    plsc.store_scatter(cdst, [pos], dv, mask=m)
                    return n + jnp.max(cum)

                return pl.loop(0, 64, init_carry=n)(grp)

            n = pl.loop(0, SBLK, init_carry=jnp.int32(0))(blk)

            for t in range(8):
                csrc[pl.ds(n + t * 16, 16)] = zi
                cdst[pl.ds(n + t * 16, 16)] = di

            nchk = (n + CH - 1) // CH
            for t in range(NRB - 1):
                @pl.when(t < nchk)
                def _():
                    fire(t, t)

            @pl.loop(0, (nchk + NRB - 1) // NRB)
            def _(i):
                for t in range(NRB):
                    k = i * NRB + t

                    @pl.when(k < nchk)
                    def _():
                        drain(t)
                        @pl.when(k + NRB - 1 < nchk)
                        def _():
                            fire(k + NRB - 1, (t + NRB - 1) % NRB)
                        _cpy_idx(dtmp, cdst, k * CH)
                        pltpu.sync_copy(rbufs[t], acc.at[dtmp], add=True)

        plsc.subcore_barrier()
        o = pl.multiple_of(sid * (RNG_G // 16), 8)
        pltpu.sync_copy(acc.at[pl.ds(o, RNG_G // 16)],
                        out_hbm.at[cid, pl.ds(r0 + o, RNG_G // 16)])
        plsc.subcore_barrier()


_aggg_call = pl.kernel(
    _aggg_body,
    out_type=jax.ShapeDtypeStruct((NC, GOUT, 128), _f32),
    mesh=_mesh,
    scratch_types=[
        pltpu.VMEM((1024,), jnp.int32),
        pltpu.VMEM((1024,), jnp.int32),
        pltpu.VMEM((NCMP,), jnp.int32),
        pltpu.VMEM((NCMP,), jnp.int32),
        pltpu.VMEM((CH,), jnp.int32),
        pltpu.VMEM((CH, 128), _f32),
        pltpu.VMEM((CH, 128), _f32),
        pltpu.VMEM((CH, 128), _f32),
        pltpu.VMEM((CH, 128), _f32),
        pltpu.SemaphoreType.DMA,
        pltpu.SemaphoreType.DMA,
        pltpu.SemaphoreType.DMA,
        pltpu.SemaphoreType.DMA,
        pltpu.VMEM_SHARED((ACC_GR, 128), _f32),
    ],
    compiler_params=_sc_params,
)


def _pairs_body(g_hbm, p_hbm, pg_hbm, pp_hbm, go_hbm, po_hbm,
                idx, rows, sem):
    w = _worker_id()
    nb = jnp.where(w < NPB - (NPB // NW) * NW, NPB // NW + 1, NPB // NW)

    @pl.loop(0, nb)
    def _(k):
        b = w + k * NW
        for tbl, ihbm, ohbm in ((g_hbm, pg_hbm, go_hbm),
                                (p_hbm, pp_hbm, po_hbm)):
            pltpu.sync_copy(ihbm.at[pl.ds(b * 8, 8)], idx)
            for h in range(2):
                for j in range(4):
                    pltpu.async_copy(tbl.at[idx.at[h * 4 + j]],
                                     rows.at[pl.ds(j * 128, 128)], sem).wait()
                o = pl.multiple_of(b * 1024 + h * 512, 8)
                pltpu.sync_copy(rows, ohbm.at[pl.ds(o, 512)])


_pairs_call = pl.kernel(
    _pairs_body,
    out_type=[jax.ShapeDtypeStruct((NPP, 128), _f32),
              jax.ShapeDtypeStruct((NPP, 128), _f32)],
    mesh=_mesh,
    scratch_types=[
        pltpu.VMEM((8, 128), jnp.int32),
        pltpu.VMEM((512, 128), _f32),
        pltpu.SemaphoreType.DMA,
    ],
)


def _dot_t(a, w):
    return lax.dot_general(a, w, dimension_numbers=(((1,), (1,)), ((), ())),
                           preferred_element_type=_f32)


def _cntsum_body(cin, o):
    c = jnp.sum(cin[...], axis=0)
    o[...] = (1.0 / jnp.maximum(c, 1.0))[:, None] * jnp.ones((1, 128), _f32)


def _cntsum(cflat, hn):
    bn = 1024
    return pl.pallas_call(
        _cntsum_body,
        grid=(hn // bn,),
        in_specs=[pl.BlockSpec((NW, bn), lambda i: (0, i))],
        out_specs=pl.BlockSpec((bn, 128), lambda i: (i, 0)),
        out_shape=jax.ShapeDtypeStruct((hn, 128), _f32),
    )(cflat.reshape(NW, hn))


def _sage_body(relu, parts, rcp, x, wl, wr, b, o):
    agg = (parts[0] + parts[1]) * rcp[...]
    r = _dot_t(agg, wl[...]) + b[...] + _dot_t(x[...], wr[...])
    o[...] = jnp.maximum(r, 0.0) if relu else r


def _sage(parts, rcp, x, wl, wr, b, relu, n):
    bn = 1000
    return pl.pallas_call(
        functools.partial(_sage_body, relu),
        grid=(n // bn,),
        in_specs=[
            pl.BlockSpec((NC, bn, 128), lambda i: (0, i, 0)),
            pl.BlockSpec((bn, 128), lambda i: (i, 0)),
            pl.BlockSpec((bn, 128), lambda i: (i, 0)),
            pl.BlockSpec((128, 128), lambda i: (0, 0)),
            pl.BlockSpec((128, 128), lambda i: (0, 0)),
            pl.BlockSpec((1, 128), lambda i: (0, 0)),
        ],
        out_specs=pl.BlockSpec((bn, 128), lambda i: (i, 0)),
        out_shape=jax.ShapeDtypeStruct((n, 128), _f32),
    )(parts, rcp, x, wl, wr, b)


def _proj_body(x, w, o):
    o[...] = _dot_t(x[...], w[...])


def _proj(x, w):
    bn = 1000
    return pl.pallas_call(
        _proj_body,
        grid=(NPNL // bn,),
        in_specs=[pl.BlockSpec((bn, 128), lambda i: (i, 0)),
                  pl.BlockSpec((128, 128), lambda i: (0, 0))],
        out_specs=pl.BlockSpec((bn, 128), lambda i: (i, 0)),
        out_shape=jax.ShapeDtypeStruct((NPNL, 128), _f32),
    )(x, w)


def _score_body(g, p, bb, o):
    o[...] = jnp.sum(g[...] * p[...], axis=1) + bb[0, 0]


def _score(gr, pr, bb):
    bn = 2048
    return pl.pallas_call(
        _score_body,
        grid=(NPP // bn,),
        in_specs=[pl.BlockSpec((bn, 128), lambda i: (i, 0)),
                  pl.BlockSpec((bn, 128), lambda i: (i, 0)),
                  pl.BlockSpec((1, 1), lambda i: (0, 0))],
        out_specs=pl.BlockSpec((bn,), lambda i: (i,)),
        out_shape=jax.ShapeDtypeStruct((NPP,), _f32),
    )(gr, pr, bb)


def _pad_idx(a, n, fill):
    a = a.astype(jnp.int32)
    return jnp.concatenate([a, jnp.full((n - a.shape[0],), fill, jnp.int32)])


def kernel(x_gene, x_panel, edge_gene, edge_panel, pairs_gene, pairs_panel,
           W_l1_p, b_l1_p, W_r1_p, W_l1_g, b_l1_g, W_r1_g,
           W_l2_p, b_l2_p, W_r2_p, W_l2_g, b_l2_g, W_r2_g, W_bil, b_bil):
    src_bf = _pad_idx(edge_gene, NEP, 0)
    dst_bf = _pad_idx(edge_panel, NEP, NPNL)
    src_cf = _pad_idx(edge_panel, NEP, 0)
    dst_cf = _pad_idx(edge_gene, NEP, NG)
    pg = _pad_idx(pairs_gene, NPP, 0).reshape(-1, 128)
    pp = _pad_idx(pairs_panel, NPP, 0).reshape(-1, 128)
    zro = jnp.zeros((ACC_GR // 16, 128), _f32)

    cpf, cgf = _counts_call(dst_bf, dst_cf)
    rcpp = _cntsum(cpf, HP_N)
    rcpg = _cntsum(cgf, HG_N)

    b1p = b_l1_p.reshape(1, 128)
    b1g = b_l1_g.reshape(1, 128)
    b2p = b_l2_p.reshape(1, 128)
    b2g = b_l2_g.reshape(1, 128)

    aggp1 = _aggp_call(x_gene, src_bf, dst_bf, zro)
    aggg1 = _aggg_call(x_panel, src_cf, dst_cf, zro)
    p1 = _sage(aggp1, rcpp[:NPNL], x_panel, W_l1_p, W_r1_p, b1p, True, NPNL)
    g1 = _sage(aggg1, rcpg[:NG], x_gene, W_l1_g, W_r1_g, b1g, True, NG)

    aggp2 = _aggp_call(g1, src_bf, dst_bf, zro)
    aggg2 = _aggg_call(p1, src_cf, dst_cf, zro)
    p2 = _sage(aggp2, rcpp[:NPNL], p1, W_l2_p, W_r2_p, b2p, False, NPNL)
    g2 = _sage(aggg2, rcpg[:NG], g1, W_l2_g, W_r2_g, b2g, False, NG)

    p2t = _proj(p2, W_bil[0])
    gr, pr = _pairs_call(g2, p2t, pg, pp)
    s = _score(gr, pr, b_bil.reshape(1, 1))
    return s[:NPAIR]
